# baseline (speedup 1.0000x reference)
import functools
import jax
import jax.numpy as jnp
from jax import lax
from jax.experimental import pallas as pl
from jax.experimental.pallas import tpu as pltpu
from jax.experimental.pallas import tpu_sc as plsc

NC, NS, LANES = 2, 16, 16
NW = NC * NS
B = 16384
GROUP = 20
D = 128
NSUB = D // LANES
CHUNK = 64
PER_W = B // NW
N_CHUNKS = PER_W // CHUNK


def _sc_body(user_hbm, item_hbm, g_hbm, u_hbm, i_hbm, j_hbm,
             s1_hbm, s2_hbm, s3_hbm,
             graw, idxg, idxu, idxi, idxj,
             acc0, acc1, eu0, eu1, ei0, ei1, ej0, ej1,
             sv1, sv2, sv3, sem0, sem1):
    cid = lax.axis_index("c")
    sid = lax.axis_index("s")
    wid = sid * NC + cid
    wbase = wid * PER_W

    accs = (acc0, acc1)
    eus = (eu0, eu1)
    eis = (ei0, ei1)
    ejs = (ej0, ej1)
    sems = (sem0, sem1)

    pltpu.sync_copy(u_hbm.at[pl.ds(wbase, PER_W)], idxu)
    pltpu.sync_copy(i_hbm.at[pl.ds(wbase, PER_W)], idxi)
    pltpu.sync_copy(j_hbm.at[pl.ds(wbase, PER_W)], idxj)

    zero16 = jnp.zeros((LANES,), jnp.float32)
    lane_id = lax.iota(jnp.int32, LANES)

    GCH = 128
    for r0 in range(0, PER_W, GCH):
        pltpu.sync_copy(g_hbm.at[pl.ds(wbase + r0, GCH), :], graw)

        @pl.loop(0, GCH // LANES)
        def transpose_loop(w):
            rows = w * LANES + lane_id
            for g in range(GROUP):
                col = plsc.load_gather(
                    graw, [rows, jnp.full((LANES,), g, jnp.int32)])
                idxg[g, pl.ds(r0 + w * LANES, LANES)] = col

    def zero_acc(b):
        @pl.loop(0, CHUNK)
        def zero_loop(e):
            for k in range(NSUB):
                accs[b][e, pl.ds(k * LANES, LANES)] = zero16

    def fire(tbase, b):
        for g in range(GROUP):
            pltpu.async_copy(
                user_hbm.at[idxg.at[g, pl.ds(tbase, CHUNK)]], accs[b],
                sems[b], add=True)
        pltpu.async_copy(user_hbm.at[idxu.at[pl.ds(tbase, CHUNK)]],
                         eus[b], sems[b])
        pltpu.async_copy(item_hbm.at[idxi.at[pl.ds(tbase, CHUNK)]],
                         eis[b], sems[b])
        pltpu.async_copy(item_hbm.at[idxj.at[pl.ds(tbase, CHUNK)]],
                         ejs[b], sems[b])

    def drain(b):
        dummy = user_hbm.at[pl.ds(0, CHUNK)]
        for _ in range(GROUP):
            pltpu.make_async_copy(dummy, accs[b], sems[b]).wait()
        pltpu.make_async_copy(dummy, eus[b], sems[b]).wait()
        pltpu.make_async_copy(dummy, eis[b], sems[b]).wait()
        pltpu.make_async_copy(dummy, ejs[b], sems[b]).wait()

    def compute(tbase, b):
        acc, eu, ei, ej = accs[b], eus[b], eis[b], ejs[b]

        @pl.loop(0, CHUNK // LANES)
        def elem_loop(w):
            o1 = zero16
            o2 = zero16
            o3 = zero16
            for m in range(LANES):
                e = w * LANES + m
                d1 = zero16
                d2 = zero16
                d3 = zero16
                for k in range(NSUB):
                    sl = pl.ds(k * LANES, LANES)
                    va = acc[e, sl]
                    vu = eu[e, sl]
                    vi = ei[e, sl]
                    vj = ej[e, sl]
                    d1 = d1 + va * vi
                    d2 = d2 + vu * vi
                    d3 = d3 + vu * vj
                msk = lane_id == m
                o1 = jnp.where(msk, jnp.sum(d1), o1)
                o2 = jnp.where(msk, jnp.sum(d2), o2)
                o3 = jnp.where(msk, jnp.sum(d3), o3)
            sl16 = pl.ds(tbase + w * LANES, LANES)
            sv1[sl16] = o1
            sv2[sl16] = o2
            sv3[sl16] = o3

    zero_acc(0)
    fire(0, 0)

    @pl.loop(0, N_CHUNKS, step=2)
    def chunk_loop(t):
        tb0 = t * CHUNK
        tb1 = tb0 + CHUNK
        zero_acc(1)
        fire(tb1, 1)
        drain(0)
        compute(tb0, 0)

        @pl.when(t + 2 < N_CHUNKS)
        def prefetch_even():
            zero_acc(0)
            fire(tb1 + CHUNK, 0)

        drain(1)
        compute(tb1, 1)

    pltpu.sync_copy(sv1, s1_hbm.at[pl.ds(wbase, PER_W)])
    pltpu.sync_copy(sv2, s2_hbm.at[pl.ds(wbase, PER_W)])
    pltpu.sync_copy(sv3, s3_hbm.at[pl.ds(wbase, PER_W)])


@jax.jit
def _sc_call(user_matrix, item_matrix, gt, u, i, j):
    fvec = jax.ShapeDtypeStruct((B,), jnp.float32)
    return pl.kernel(
        _sc_body,
        out_type=(fvec, fvec, fvec),
        mesh=plsc.VectorSubcoreMesh(
            core_axis_name="c", subcore_axis_name="s",
            num_cores=NC, num_subcores=NS),
        compiler_params=pltpu.CompilerParams(needs_layout_passes=False),
        scratch_types=[
            pltpu.VMEM((128, GROUP), jnp.int32),
            pltpu.VMEM((GROUP, PER_W), jnp.int32),
            pltpu.VMEM((PER_W,), jnp.int32),
            pltpu.VMEM((PER_W,), jnp.int32),
            pltpu.VMEM((PER_W,), jnp.int32),
            pltpu.VMEM((CHUNK, D), jnp.float32),
            pltpu.VMEM((CHUNK, D), jnp.float32),
            pltpu.VMEM((CHUNK, D), jnp.float32),
            pltpu.VMEM((CHUNK, D), jnp.float32),
            pltpu.VMEM((CHUNK, D), jnp.float32),
            pltpu.VMEM((CHUNK, D), jnp.float32),
            pltpu.VMEM((CHUNK, D), jnp.float32),
            pltpu.VMEM((CHUNK, D), jnp.float32),
            pltpu.VMEM((PER_W,), jnp.float32),
            pltpu.VMEM((PER_W,), jnp.float32),
            pltpu.VMEM((PER_W,), jnp.float32),
            pltpu.SemaphoreType.DMA,
            pltpu.SemaphoreType.DMA,
        ],
    )(user_matrix, item_matrix, gt, u, i, j)


def _tc_body(s1_ref, s2_ref, s3_ref, ratio_ref, out_ref):
    r_gi = s1_ref[...] * (1.0 / GROUP)
    r_ui = s2_ref[...]
    r_uj = s3_ref[...]
    ratio = ratio_ref[0]
    r_gui = ratio * (r_gi - r_ui) + r_ui
    x = r_gui - r_uj
    out_ref[0, 0] = -jnp.sum(jnp.log(jax.nn.sigmoid(x)))


@jax.jit
def _tc_call(s1, s2, s3, ratio):
    return pl.pallas_call(
        _tc_body,
        out_shape=jax.ShapeDtypeStruct((1, 1), jnp.float32),
        in_specs=[
            pl.BlockSpec(memory_space=pltpu.VMEM),
            pl.BlockSpec(memory_space=pltpu.VMEM),
            pl.BlockSpec(memory_space=pltpu.VMEM),
            pl.BlockSpec(memory_space=pltpu.SMEM),
        ],
        out_specs=pl.BlockSpec(memory_space=pltpu.SMEM),
    )(s1, s2, s3, ratio)


def kernel(user_matrix, item_matrix, u, i, j, G, ratio):
    u = u.astype(jnp.int32)
    i = i.astype(jnp.int32)
    j = j.astype(jnp.int32)
    g = G.astype(jnp.int32)
    s1, s2, s3 = _sc_call(user_matrix, item_matrix, g, u, i, j)
    loss = _tc_call(s1.reshape(128, 128), s2.reshape(128, 128),
                    s3.reshape(128, 128), ratio.reshape(1))
    return loss[0, 0]


# speedup vs baseline: 1.0956x; 1.0956x over previous
import functools
import jax
import jax.numpy as jnp
from jax import lax
from jax.experimental import pallas as pl
from jax.experimental.pallas import tpu as pltpu
from jax.experimental.pallas import tpu_sc as plsc

NC, NS, LANES = 2, 16, 16
NW = NC * NS
B = 16384
GROUP = 20
D = 128
NSUB = D // LANES
CHUNK = 64
PER_W = B // NW
N_CHUNKS = PER_W // CHUNK


def _sc_body(user_hbm, item_hbm, g_hbm, u_hbm, i_hbm, j_hbm,
             s1_hbm, s2_hbm, s3_hbm,
             graw0, graw1, idxg, idxu, idxi, idxj,
             acc0, acc1, eu0, eu1, ei0, ei1, ej0, ej1,
             sv1, sv2, sv3, sem0, sem1, gsem):
    cid = lax.axis_index("c")
    sid = lax.axis_index("s")
    wid = sid * NC + cid
    wbase = wid * PER_W

    accs = (acc0, acc1)
    eus = (eu0, eu1)
    eis = (ei0, ei1)
    ejs = (ej0, ej1)
    sems = (sem0, sem1)

    pltpu.sync_copy(u_hbm.at[pl.ds(wbase, PER_W)], idxu)
    pltpu.sync_copy(i_hbm.at[pl.ds(wbase, PER_W)], idxi)
    pltpu.sync_copy(j_hbm.at[pl.ds(wbase, PER_W)], idxj)

    zero16 = jnp.zeros((LANES,), jnp.float32)
    lane_id = lax.iota(jnp.int32, LANES)
    graws = (graw0, graw1)

    def fire_g(c, p):
        pltpu.async_copy(
            g_hbm.at[pl.ds(wbase + c * CHUNK, CHUNK), :], graws[p], gsem)

    def wait_g(p):
        pltpu.make_async_copy(
            g_hbm.at[pl.ds(0, CHUNK), :], graws[p], gsem).wait()

    def trans_g(c, p):
        tb = c * CHUNK

        @pl.loop(0, CHUNK // LANES)
        def transpose_loop(w):
            rows = w * LANES + lane_id
            for g in range(GROUP):
                col = plsc.load_gather(
                    graws[p], [rows, jnp.full((LANES,), g, jnp.int32)])
                idxg[g, pl.ds(tb + w * LANES, LANES)] = col

    def zero_acc(b):
        @pl.loop(0, CHUNK)
        def zero_loop(e):
            for k in range(NSUB):
                accs[b][e, pl.ds(k * LANES, LANES)] = zero16

    def fire(tbase, b):
        for g in range(GROUP):
            pltpu.async_copy(
                user_hbm.at[idxg.at[g, pl.ds(tbase, CHUNK)]], accs[b],
                sems[b], add=True)
        pltpu.async_copy(user_hbm.at[idxu.at[pl.ds(tbase, CHUNK)]],
                         eus[b], sems[b])
        pltpu.async_copy(item_hbm.at[idxi.at[pl.ds(tbase, CHUNK)]],
                         eis[b], sems[b])
        pltpu.async_copy(item_hbm.at[idxj.at[pl.ds(tbase, CHUNK)]],
                         ejs[b], sems[b])

    def drain(b):
        dummy = user_hbm.at[pl.ds(0, CHUNK)]
        for _ in range(GROUP):
            pltpu.make_async_copy(dummy, accs[b], sems[b]).wait()
        pltpu.make_async_copy(dummy, eus[b], sems[b]).wait()
        pltpu.make_async_copy(dummy, eis[b], sems[b]).wait()
        pltpu.make_async_copy(dummy, ejs[b], sems[b]).wait()

    def compute(tbase, b):
        acc, eu, ei, ej = accs[b], eus[b], eis[b], ejs[b]

        @pl.loop(0, CHUNK // LANES)
        def elem_loop(w):
            o1 = zero16
            o2 = zero16
            o3 = zero16
            for m in range(LANES):
                e = w * LANES + m
                d1 = zero16
                d2 = zero16
                d3 = zero16
                for k in range(NSUB):
                    sl = pl.ds(k * LANES, LANES)
                    va = acc[e, sl]
                    vu = eu[e, sl]
                    vi = ei[e, sl]
                    vj = ej[e, sl]
                    d1 = d1 + va * vi
                    d2 = d2 + vu * vi
                    d3 = d3 + vu * vj
                msk = lane_id == m
                o1 = jnp.where(msk, jnp.sum(d1), o1)
                o2 = jnp.where(msk, jnp.sum(d2), o2)
                o3 = jnp.where(msk, jnp.sum(d3), o3)
            sl16 = pl.ds(tbase + w * LANES, LANES)
            sv1[sl16] = o1
            sv2[sl16] = o2
            sv3[sl16] = o3

    fire_g(0, 0)
    wait_g(0)
    trans_g(0, 0)
    fire_g(1, 1)
    zero_acc(0)
    fire(0, 0)

    @pl.loop(0, N_CHUNKS, step=2)
    def chunk_loop(t):
        tb0 = t * CHUNK
        tb1 = tb0 + CHUNK
        wait_g(1)
        trans_g(t + 1, 1)

        @pl.when(t + 2 < N_CHUNKS)
        def stage_even():
            fire_g(t + 2, 0)

        zero_acc(1)
        fire(tb1, 1)
        drain(0)
        compute(tb0, 0)

        @pl.when(t + 2 < N_CHUNKS)
        def prefetch_even():
            wait_g(0)
            trans_g(t + 2, 0)

            @pl.when(t + 3 < N_CHUNKS)
            def stage_odd():
                fire_g(t + 3, 1)

            zero_acc(0)
            fire(tb1 + CHUNK, 0)

        drain(1)
        compute(tb1, 1)

    pltpu.sync_copy(sv1, s1_hbm.at[pl.ds(wbase, PER_W)])
    pltpu.sync_copy(sv2, s2_hbm.at[pl.ds(wbase, PER_W)])
    pltpu.sync_copy(sv3, s3_hbm.at[pl.ds(wbase, PER_W)])


@jax.jit
def _sc_call(user_matrix, item_matrix, gt, u, i, j):
    fvec = jax.ShapeDtypeStruct((B,), jnp.float32)
    return pl.kernel(
        _sc_body,
        out_type=(fvec, fvec, fvec),
        mesh=plsc.VectorSubcoreMesh(
            core_axis_name="c", subcore_axis_name="s",
            num_cores=NC, num_subcores=NS),
        compiler_params=pltpu.CompilerParams(needs_layout_passes=False),
        scratch_types=[
            pltpu.VMEM((CHUNK, GROUP), jnp.int32),
            pltpu.VMEM((CHUNK, GROUP), jnp.int32),
            pltpu.VMEM((GROUP, PER_W), jnp.int32),
            pltpu.VMEM((PER_W,), jnp.int32),
            pltpu.VMEM((PER_W,), jnp.int32),
            pltpu.VMEM((PER_W,), jnp.int32),
            pltpu.VMEM((CHUNK, D), jnp.float32),
            pltpu.VMEM((CHUNK, D), jnp.float32),
            pltpu.VMEM((CHUNK, D), jnp.float32),
            pltpu.VMEM((CHUNK, D), jnp.float32),
            pltpu.VMEM((CHUNK, D), jnp.float32),
            pltpu.VMEM((CHUNK, D), jnp.float32),
            pltpu.VMEM((CHUNK, D), jnp.float32),
            pltpu.VMEM((CHUNK, D), jnp.float32),
            pltpu.VMEM((PER_W,), jnp.float32),
            pltpu.VMEM((PER_W,), jnp.float32),
            pltpu.VMEM((PER_W,), jnp.float32),
            pltpu.SemaphoreType.DMA,
            pltpu.SemaphoreType.DMA,
            pltpu.SemaphoreType.DMA,
        ],
    )(user_matrix, item_matrix, gt, u, i, j)


def _tc_body(s1_ref, s2_ref, s3_ref, ratio_ref, out_ref):
    r_gi = s1_ref[...] * (1.0 / GROUP)
    r_ui = s2_ref[...]
    r_uj = s3_ref[...]
    ratio = ratio_ref[0]
    r_gui = ratio * (r_gi - r_ui) + r_ui
    x = r_gui - r_uj
    out_ref[0, 0] = -jnp.sum(jnp.log(jax.nn.sigmoid(x)))


@jax.jit
def _tc_call(s1, s2, s3, ratio):
    return pl.pallas_call(
        _tc_body,
        out_shape=jax.ShapeDtypeStruct((1, 1), jnp.float32),
        in_specs=[
            pl.BlockSpec(memory_space=pltpu.VMEM),
            pl.BlockSpec(memory_space=pltpu.VMEM),
            pl.BlockSpec(memory_space=pltpu.VMEM),
            pl.BlockSpec(memory_space=pltpu.SMEM),
        ],
        out_specs=pl.BlockSpec(memory_space=pltpu.SMEM),
    )(s1, s2, s3, ratio)


def kernel(user_matrix, item_matrix, u, i, j, G, ratio):
    u = u.astype(jnp.int32)
    i = i.astype(jnp.int32)
    j = j.astype(jnp.int32)
    g = G.astype(jnp.int32)
    s1, s2, s3 = _sc_call(user_matrix, item_matrix, g, u, i, j)
    loss = _tc_call(s1.reshape(128, 128), s2.reshape(128, 128),
                    s3.reshape(128, 128), ratio.reshape(1))
    return loss[0, 0]


# speedup vs baseline: 1.1597x; 1.0585x over previous
import functools
import jax
import jax.numpy as jnp
from jax import lax
from jax.experimental import pallas as pl
from jax.experimental.pallas import tpu as pltpu
from jax.experimental.pallas import tpu_sc as plsc

NC, NS, LANES = 2, 16, 16
NW = NC * NS
B = 16384
GROUP = 20
D = 128
NSUB = D // LANES
CHUNK = 64
PER_W = B // NW
N_CHUNKS = PER_W // CHUNK


def _sc_body(user_hbm, item_hbm, idx_hbm,
             s1_hbm, s2_hbm, s3_hbm,
             idxg,
             acc0, acc1, eu0, eu1, ei0, ei1, ej0, ej1,
             sv1, sv2, sv3, sem0, sem1):
    cid = lax.axis_index("c")
    sid = lax.axis_index("s")
    wid = sid * NC + cid
    wbase = wid * PER_W

    accs = (acc0, acc1)
    eus = (eu0, eu1)
    eis = (ei0, ei1)
    ejs = (ej0, ej1)
    sems = (sem0, sem1)

    pltpu.sync_copy(idx_hbm.at[:, pl.ds(wbase, PER_W)], idxg)

    zero16 = jnp.zeros((LANES,), jnp.float32)
    lane_id = lax.iota(jnp.int32, LANES)

    def zero_acc(b):
        @pl.loop(0, CHUNK)
        def zero_loop(e):
            for k in range(NSUB):
                accs[b][e, pl.ds(k * LANES, LANES)] = zero16

    def fire(tbase, b):
        for g in range(GROUP):
            pltpu.async_copy(
                user_hbm.at[idxg.at[g, pl.ds(tbase, CHUNK)]], accs[b],
                sems[b], add=True)
        pltpu.async_copy(user_hbm.at[idxg.at[GROUP, pl.ds(tbase, CHUNK)]],
                         eus[b], sems[b])
        pltpu.async_copy(item_hbm.at[idxg.at[GROUP + 1,
                                             pl.ds(tbase, CHUNK)]],
                         eis[b], sems[b])
        pltpu.async_copy(item_hbm.at[idxg.at[GROUP + 2,
                                             pl.ds(tbase, CHUNK)]],
                         ejs[b], sems[b])

    def drain(b):
        dummy = user_hbm.at[pl.ds(0, CHUNK)]
        for _ in range(GROUP):
            pltpu.make_async_copy(dummy, accs[b], sems[b]).wait()
        pltpu.make_async_copy(dummy, eus[b], sems[b]).wait()
        pltpu.make_async_copy(dummy, eis[b], sems[b]).wait()
        pltpu.make_async_copy(dummy, ejs[b], sems[b]).wait()

    def compute(tbase, b):
        acc, eu, ei, ej = accs[b], eus[b], eis[b], ejs[b]

        @pl.loop(0, CHUNK // LANES)
        def elem_loop(w):
            o1 = zero16
            o2 = zero16
            o3 = zero16
            for m in range(LANES):
                e = w * LANES + m
                d1 = zero16
                d2 = zero16
                d3 = zero16
                for k in range(NSUB):
                    sl = pl.ds(k * LANES, LANES)
                    va = acc[e, sl]
                    vu = eu[e, sl]
                    vi = ei[e, sl]
                    vj = ej[e, sl]
                    d1 = d1 + va * vi
                    d2 = d2 + vu * vi
                    d3 = d3 + vu * vj
                msk = lane_id == m
                o1 = jnp.where(msk, jnp.sum(d1), o1)
                o2 = jnp.where(msk, jnp.sum(d2), o2)
                o3 = jnp.where(msk, jnp.sum(d3), o3)
            sl16 = pl.ds(tbase + w * LANES, LANES)
            sv1[sl16] = o1
            sv2[sl16] = o2
            sv3[sl16] = o3

    zero_acc(0)
    fire(0, 0)

    @pl.loop(0, N_CHUNKS, step=2)
    def chunk_loop(t):
        tb0 = t * CHUNK
        tb1 = tb0 + CHUNK
        zero_acc(1)
        fire(tb1, 1)
        drain(0)
        compute(tb0, 0)

        @pl.when(t + 2 < N_CHUNKS)
        def prefetch_even():
            zero_acc(0)
            fire(tb1 + CHUNK, 0)

        drain(1)
        compute(tb1, 1)

    d1 = pltpu.async_copy(sv1, s1_hbm.at[pl.ds(wbase, PER_W)], sem0)
    d2 = pltpu.async_copy(sv2, s2_hbm.at[pl.ds(wbase, PER_W)], sem0)
    d3 = pltpu.async_copy(sv3, s3_hbm.at[pl.ds(wbase, PER_W)], sem0)
    d1.wait()
    d2.wait()
    d3.wait()


@jax.jit
def _sc_call(user_matrix, item_matrix, idx_all):
    fvec = jax.ShapeDtypeStruct((B,), jnp.float32)
    return pl.kernel(
        _sc_body,
        out_type=(fvec, fvec, fvec),
        mesh=plsc.VectorSubcoreMesh(
            core_axis_name="c", subcore_axis_name="s",
            num_cores=NC, num_subcores=NS),
        compiler_params=pltpu.CompilerParams(needs_layout_passes=False),
        scratch_types=[
            pltpu.VMEM((GROUP + 3, PER_W), jnp.int32),
            pltpu.VMEM((CHUNK, D), jnp.float32),
            pltpu.VMEM((CHUNK, D), jnp.float32),
            pltpu.VMEM((CHUNK, D), jnp.float32),
            pltpu.VMEM((CHUNK, D), jnp.float32),
            pltpu.VMEM((CHUNK, D), jnp.float32),
            pltpu.VMEM((CHUNK, D), jnp.float32),
            pltpu.VMEM((CHUNK, D), jnp.float32),
            pltpu.VMEM((CHUNK, D), jnp.float32),
            pltpu.VMEM((PER_W,), jnp.float32),
            pltpu.VMEM((PER_W,), jnp.float32),
            pltpu.VMEM((PER_W,), jnp.float32),
            pltpu.SemaphoreType.DMA,
            pltpu.SemaphoreType.DMA,
        ],
    )(user_matrix, item_matrix, idx_all)


def _tc_body(s1_ref, s2_ref, s3_ref, ratio_ref, out_ref):
    r_gi = s1_ref[...] * (1.0 / GROUP)
    r_ui = s2_ref[...]
    r_uj = s3_ref[...]
    ratio = ratio_ref[0]
    r_gui = ratio * (r_gi - r_ui) + r_ui
    x = r_gui - r_uj
    out_ref[0, 0] = -jnp.sum(jnp.log(jax.nn.sigmoid(x)))


@jax.jit
def _tc_call(s1, s2, s3, ratio):
    return pl.pallas_call(
        _tc_body,
        out_shape=jax.ShapeDtypeStruct((1, 1), jnp.float32),
        in_specs=[
            pl.BlockSpec(memory_space=pltpu.VMEM),
            pl.BlockSpec(memory_space=pltpu.VMEM),
            pl.BlockSpec(memory_space=pltpu.VMEM),
            pl.BlockSpec(memory_space=pltpu.SMEM),
        ],
        out_specs=pl.BlockSpec(memory_space=pltpu.SMEM),
    )(s1, s2, s3, ratio)


def kernel(user_matrix, item_matrix, u, i, j, G, ratio):
    idx_all = jnp.concatenate(
        [G.astype(jnp.int32).T,
         u.astype(jnp.int32)[None],
         i.astype(jnp.int32)[None],
         j.astype(jnp.int32)[None]], axis=0)
    s1, s2, s3 = _sc_call(user_matrix, item_matrix, idx_all)
    loss = _tc_call(s1.reshape(128, 128), s2.reshape(128, 128),
                    s3.reshape(128, 128), ratio.reshape(1))
    return loss[0, 0]


# speedup vs baseline: 1.1693x; 1.0083x over previous
import functools
import jax
import jax.numpy as jnp
from jax import lax
from jax.experimental import pallas as pl
from jax.experimental.pallas import tpu as pltpu
from jax.experimental.pallas import tpu_sc as plsc

NC, NS, LANES = 2, 16, 16
NW = NC * NS
B = 16384
GROUP = 20
D = 128
NSUB = D // LANES
CHUNK = 64
PER_W = B // NW
N_CHUNKS = PER_W // CHUNK


def _sc_body(user_hbm, item_hbm, idx_hbm,
             s1_hbm, s2_hbm, s3_hbm,
             idxg,
             acc0, acc1, eu0, eu1, ei0, ei1, ej0, ej1,
             sv1, sv2, sv3, sem0, sem1):
    cid = lax.axis_index("c")
    sid = lax.axis_index("s")
    wid = sid * NC + cid
    wbase = wid * PER_W

    accs = (acc0, acc1)
    eus = (eu0, eu1)
    eis = (ei0, ei1)
    ejs = (ej0, ej1)
    sems = (sem0, sem1)

    pltpu.sync_copy(idx_hbm.at[:, pl.ds(wbase, PER_W)], idxg)

    zero16 = jnp.zeros((LANES,), jnp.float32)
    lane_id = lax.iota(jnp.int32, LANES)

    def zero_acc(b):
        @pl.loop(0, CHUNK)
        def zero_loop(e):
            for k in range(NSUB):
                accs[b][e, pl.ds(k * LANES, LANES)] = zero16

    def fire(tbase, b):
        for g in range(GROUP):
            pltpu.async_copy(
                user_hbm.at[idxg.at[g, pl.ds(tbase, CHUNK)]], accs[b],
                sems[b], add=True)
        pltpu.async_copy(user_hbm.at[idxg.at[GROUP, pl.ds(tbase, CHUNK)]],
                         eus[b], sems[b])
        pltpu.async_copy(item_hbm.at[idxg.at[GROUP + 1,
                                             pl.ds(tbase, CHUNK)]],
                         eis[b], sems[b])
        pltpu.async_copy(item_hbm.at[idxg.at[GROUP + 2,
                                             pl.ds(tbase, CHUNK)]],
                         ejs[b], sems[b])

    def drain(b):
        dummy = user_hbm.at[pl.ds(0, CHUNK)]
        for _ in range(GROUP):
            pltpu.make_async_copy(dummy, accs[b], sems[b]).wait()
        pltpu.make_async_copy(dummy, eus[b], sems[b]).wait()
        pltpu.make_async_copy(dummy, eis[b], sems[b]).wait()
        pltpu.make_async_copy(dummy, ejs[b], sems[b]).wait()

    def compute(tbase, b):
        acc, eu, ei, ej = accs[b], eus[b], eis[b], ejs[b]

        @pl.loop(0, CHUNK // LANES)
        def elem_loop(w):
            o1 = zero16
            o2 = zero16
            o3 = zero16
            for m in range(LANES):
                e = w * LANES + m
                d1 = zero16
                d2 = zero16
                d3 = zero16
                for k in range(NSUB):
                    sl = pl.ds(k * LANES, LANES)
                    va = acc[e, sl]
                    vu = eu[e, sl]
                    vi = ei[e, sl]
                    vj = ej[e, sl]
                    d1 = d1 + va * vi
                    d2 = d2 + vu * vi
                    d3 = d3 + vu * vj
                    acc[e, sl] = zero16
                msk = lane_id == m
                o1 = jnp.where(msk, jnp.sum(d1), o1)
                o2 = jnp.where(msk, jnp.sum(d2), o2)
                o3 = jnp.where(msk, jnp.sum(d3), o3)
            sl16 = pl.ds(tbase + w * LANES, LANES)
            sv1[sl16] = o1
            sv2[sl16] = o2
            sv3[sl16] = o3

    zero_acc(0)
    fire(0, 0)
    zero_acc(1)

    @pl.loop(0, N_CHUNKS, step=2)
    def chunk_loop(t):
        tb0 = t * CHUNK
        tb1 = tb0 + CHUNK
        fire(tb1, 1)
        drain(0)
        compute(tb0, 0)

        @pl.when(t + 2 < N_CHUNKS)
        def prefetch_even():
            fire(tb1 + CHUNK, 0)

        drain(1)
        compute(tb1, 1)

    d1 = pltpu.async_copy(sv1, s1_hbm.at[pl.ds(wbase, PER_W)], sem0)
    d2 = pltpu.async_copy(sv2, s2_hbm.at[pl.ds(wbase, PER_W)], sem0)
    d3 = pltpu.async_copy(sv3, s3_hbm.at[pl.ds(wbase, PER_W)], sem0)
    d1.wait()
    d2.wait()
    d3.wait()


@jax.jit
def _sc_call(user_matrix, item_matrix, idx_all):
    fvec = jax.ShapeDtypeStruct((B,), jnp.float32)
    return pl.kernel(
        _sc_body,
        out_type=(fvec, fvec, fvec),
        mesh=plsc.VectorSubcoreMesh(
            core_axis_name="c", subcore_axis_name="s",
            num_cores=NC, num_subcores=NS),
        compiler_params=pltpu.CompilerParams(needs_layout_passes=False),
        scratch_types=[
            pltpu.VMEM((GROUP + 3, PER_W), jnp.int32),
            pltpu.VMEM((CHUNK, D), jnp.float32),
            pltpu.VMEM((CHUNK, D), jnp.float32),
            pltpu.VMEM((CHUNK, D), jnp.float32),
            pltpu.VMEM((CHUNK, D), jnp.float32),
            pltpu.VMEM((CHUNK, D), jnp.float32),
            pltpu.VMEM((CHUNK, D), jnp.float32),
            pltpu.VMEM((CHUNK, D), jnp.float32),
            pltpu.VMEM((CHUNK, D), jnp.float32),
            pltpu.VMEM((PER_W,), jnp.float32),
            pltpu.VMEM((PER_W,), jnp.float32),
            pltpu.VMEM((PER_W,), jnp.float32),
            pltpu.SemaphoreType.DMA,
            pltpu.SemaphoreType.DMA,
        ],
    )(user_matrix, item_matrix, idx_all)


def _tc_body(s1_ref, s2_ref, s3_ref, ratio_ref, out_ref):
    r_gi = s1_ref[...] * (1.0 / GROUP)
    r_ui = s2_ref[...]
    r_uj = s3_ref[...]
    ratio = ratio_ref[0]
    r_gui = ratio * (r_gi - r_ui) + r_ui
    x = r_gui - r_uj
    out_ref[0, 0] = -jnp.sum(jnp.log(jax.nn.sigmoid(x)))


@jax.jit
def _tc_call(s1, s2, s3, ratio):
    return pl.pallas_call(
        _tc_body,
        out_shape=jax.ShapeDtypeStruct((1, 1), jnp.float32),
        in_specs=[
            pl.BlockSpec(memory_space=pltpu.VMEM),
            pl.BlockSpec(memory_space=pltpu.VMEM),
            pl.BlockSpec(memory_space=pltpu.VMEM),
            pl.BlockSpec(memory_space=pltpu.SMEM),
        ],
        out_specs=pl.BlockSpec(memory_space=pltpu.SMEM),
    )(s1, s2, s3, ratio)


def kernel(user_matrix, item_matrix, u, i, j, G, ratio):
    idx_all = jnp.concatenate(
        [G.astype(jnp.int32).T,
         u.astype(jnp.int32)[None],
         i.astype(jnp.int32)[None],
         j.astype(jnp.int32)[None]], axis=0)
    s1, s2, s3 = _sc_call(user_matrix, item_matrix, idx_all)
    loss = _tc_call(s1.reshape(128, 128), s2.reshape(128, 128),
                    s3.reshape(128, 128), ratio.reshape(1))
    return loss[0, 0]


# speedup vs baseline: 1.1837x; 1.0122x over previous
import functools
import jax
import jax.numpy as jnp
from jax import lax
from jax.experimental import pallas as pl
from jax.experimental.pallas import tpu as pltpu
from jax.experimental.pallas import tpu_sc as plsc

NC, NS, LANES = 2, 16, 16
NW = NC * NS
B = 16384
GROUP = 20
D = 128
NSUB = D // LANES
CHUNK = 32
NBUF = 4
PER_W = B // NW
N_CHUNKS = PER_W // CHUNK


def _sc_body(*refs):
    (user_hbm, item_hbm, idx_hbm, s1_hbm, s2_hbm, s3_hbm) = refs[:6]
    idxg = refs[6]
    bufs = refs[7:7 + 4 * NBUF]
    accs = bufs[0::4]
    eus = bufs[1::4]
    eis = bufs[2::4]
    ejs = bufs[3::4]
    sv1, sv2, sv3 = refs[7 + 4 * NBUF:10 + 4 * NBUF]
    sems = refs[10 + 4 * NBUF:]

    cid = lax.axis_index("c")
    sid = lax.axis_index("s")
    wid = sid * NC + cid
    wbase = wid * PER_W

    pltpu.sync_copy(idx_hbm.at[:, pl.ds(wbase, PER_W)], idxg)

    zero16 = jnp.zeros((LANES,), jnp.float32)
    lane_id = lax.iota(jnp.int32, LANES)

    def zero_acc(b):
        @pl.loop(0, CHUNK)
        def zero_loop(e):
            for k in range(NSUB):
                accs[b][e, pl.ds(k * LANES, LANES)] = zero16

    def fire(tbase, b):
        for g in range(GROUP):
            pltpu.async_copy(
                user_hbm.at[idxg.at[g, pl.ds(tbase, CHUNK)]], accs[b],
                sems[b], add=True)
        pltpu.async_copy(user_hbm.at[idxg.at[GROUP, pl.ds(tbase, CHUNK)]],
                         eus[b], sems[b])
        pltpu.async_copy(item_hbm.at[idxg.at[GROUP + 1,
                                             pl.ds(tbase, CHUNK)]],
                         eis[b], sems[b])
        pltpu.async_copy(item_hbm.at[idxg.at[GROUP + 2,
                                             pl.ds(tbase, CHUNK)]],
                         ejs[b], sems[b])

    def drain(b):
        dummy = user_hbm.at[pl.ds(0, CHUNK)]
        for _ in range(GROUP):
            pltpu.make_async_copy(dummy, accs[b], sems[b]).wait()
        pltpu.make_async_copy(dummy, eus[b], sems[b]).wait()
        pltpu.make_async_copy(dummy, eis[b], sems[b]).wait()
        pltpu.make_async_copy(dummy, ejs[b], sems[b]).wait()

    def compute(tbase, b):
        acc, eu, ei, ej = accs[b], eus[b], eis[b], ejs[b]

        @pl.loop(0, CHUNK // LANES)
        def elem_loop(w):
            o1 = zero16
            o2 = zero16
            o3 = zero16
            for m in range(LANES):
                e = w * LANES + m
                d1 = zero16
                d2 = zero16
                d3 = zero16
                for k in range(NSUB):
                    sl = pl.ds(k * LANES, LANES)
                    va = acc[e, sl]
                    vu = eu[e, sl]
                    vi = ei[e, sl]
                    vj = ej[e, sl]
                    d1 = d1 + va * vi
                    d2 = d2 + vu * vi
                    d3 = d3 + vu * vj
                    acc[e, sl] = zero16
                msk = lane_id == m
                o1 = jnp.where(msk, jnp.sum(d1), o1)
                o2 = jnp.where(msk, jnp.sum(d2), o2)
                o3 = jnp.where(msk, jnp.sum(d3), o3)
            sl16 = pl.ds(tbase + w * LANES, LANES)
            sv1[sl16] = o1
            sv2[sl16] = o2
            sv3[sl16] = o3

    for b in range(NBUF):
        zero_acc(b)
    for c in range(NBUF - 1):
        fire(c * CHUNK, c)

    @pl.loop(0, N_CHUNKS, step=NBUF)
    def chunk_loop(t):
        for b in range(NBUF):
            c = t + b
            nxt = c + NBUF - 1

            @pl.when(nxt < N_CHUNKS)
            def prefetch():
                fire(nxt * CHUNK, (b - 1) % NBUF)

            drain(b)
            compute(c * CHUNK, b)

    d1 = pltpu.async_copy(sv1, s1_hbm.at[pl.ds(wbase, PER_W)], sems[0])
    d2 = pltpu.async_copy(sv2, s2_hbm.at[pl.ds(wbase, PER_W)], sems[0])
    d3 = pltpu.async_copy(sv3, s3_hbm.at[pl.ds(wbase, PER_W)], sems[0])
    d1.wait()
    d2.wait()
    d3.wait()


@jax.jit
def _sc_call(user_matrix, item_matrix, idx_all):
    fvec = jax.ShapeDtypeStruct((B,), jnp.float32)
    return pl.kernel(
        _sc_body,
        out_type=(fvec, fvec, fvec),
        mesh=plsc.VectorSubcoreMesh(
            core_axis_name="c", subcore_axis_name="s",
            num_cores=NC, num_subcores=NS),
        compiler_params=pltpu.CompilerParams(needs_layout_passes=False),
        scratch_types=(
            [pltpu.VMEM((GROUP + 3, PER_W), jnp.int32)]
            + [pltpu.VMEM((CHUNK, D), jnp.float32)
               for _ in range(4 * NBUF)]
            + [pltpu.VMEM((PER_W,), jnp.float32)
               for _ in range(3)]
            + [pltpu.SemaphoreType.DMA for _ in range(NBUF)]
        ),
    )(user_matrix, item_matrix, idx_all)


def _tc_body(s1_ref, s2_ref, s3_ref, ratio_ref, out_ref):
    r_gi = s1_ref[...] * (1.0 / GROUP)
    r_ui = s2_ref[...]
    r_uj = s3_ref[...]
    ratio = ratio_ref[0]
    r_gui = ratio * (r_gi - r_ui) + r_ui
    x = r_gui - r_uj
    out_ref[0, 0] = -jnp.sum(jnp.log(jax.nn.sigmoid(x)))


@jax.jit
def _tc_call(s1, s2, s3, ratio):
    return pl.pallas_call(
        _tc_body,
        out_shape=jax.ShapeDtypeStruct((1, 1), jnp.float32),
        in_specs=[
            pl.BlockSpec(memory_space=pltpu.VMEM),
            pl.BlockSpec(memory_space=pltpu.VMEM),
            pl.BlockSpec(memory_space=pltpu.VMEM),
            pl.BlockSpec(memory_space=pltpu.SMEM),
        ],
        out_specs=pl.BlockSpec(memory_space=pltpu.SMEM),
    )(s1, s2, s3, ratio)


def kernel(user_matrix, item_matrix, u, i, j, G, ratio):
    idx_all = jnp.concatenate(
        [G.astype(jnp.int32).T,
         u.astype(jnp.int32)[None],
         i.astype(jnp.int32)[None],
         j.astype(jnp.int32)[None]], axis=0)
    s1, s2, s3 = _sc_call(user_matrix, item_matrix, idx_all)
    loss = _tc_call(s1.reshape(128, 128), s2.reshape(128, 128),
                    s3.reshape(128, 128), ratio.reshape(1))
    return loss[0, 0]
